# baseline (speedup 1.0000x reference)
import jax
import jax.numpy as jnp
from jax import lax
from jax.experimental import pallas as pl
from jax.experimental.pallas import tpu as pltpu
from jax.experimental.pallas import tpu_sc as plsc

_IMAGE = 64
_ZNEAR = 0.05
_NF = 2000
_NV = 1200
_FPAD = 2048
_NPIX = _IMAGE * _IMAGE
_NC = 2
_NS = 16
_NW = _NC * _NS
_FPW = _FPAD // _NW
_PPW = _NPIX // _NW
_BIG = 1e38
_L = 16
_VPAD = 1216


def _iota16():
    return lax.broadcasted_iota(jnp.int32, (_L,), 0)


_FC = 256
_NCHUNK = _FPAD // _FC
_PT = 8
_NSUB = _IMAGE // _PT


_ROWS_PER_STEP = 8
_TCSTEPS = _IMAGE // _ROWS_PER_STEP


def _raster_body(vcoord_ref, fidx_ref, idx_ref, zmin_ref, fd_ref):
    g = pl.program_id(0)

    @pl.when(g == 0)
    def _prologue():
        svl = lax.broadcasted_iota(jnp.int32, (_VPAD, _FPAD), 0)
        dn = (((1,), (0,)), ((), ()))
        for c3 in range(3):
            ohb = (svl == fidx_ref[c3:c3 + 1, :]).astype(jnp.bfloat16)
            for k3 in range(3):
                v = vcoord_ref[k3:k3 + 1, :]
                vh = v.astype(jnp.bfloat16)
                vhf = vh.astype(jnp.float32)
                vm = (v - vhf).astype(jnp.bfloat16)
                vmf = vm.astype(jnp.float32)
                vl = (v - vhf - vmf).astype(jnp.bfloat16)
                row = (lax.dot_general(vh, ohb, dn,
                                       preferred_element_type=jnp.float32)
                       + lax.dot_general(vm, ohb, dn,
                                         preferred_element_type=jnp.float32)
                       + lax.dot_general(vl, ohb, dn,
                                         preferred_element_type=jnp.float32))
                fd_ref[c3 * 3 + k3:c3 * 3 + k3 + 1, :] = row

    gf = (_ROWS_PER_STEP * g).astype(jnp.float32)
    PYs = [1.0 - (gf + (r + 0.5)) * (2.0 / _IMAGE)
           for r in range(_ROWS_PER_STEP)]

    rows = []
    for c in range(_NCHUNK):
        sl = pl.ds(c * _FC, _FC)
        x0, y0, z0 = fd_ref[0:1, sl], fd_ref[1:2, sl], fd_ref[2:3, sl]
        x1, y1, z1 = fd_ref[3:4, sl], fd_ref[4:5, sl], fd_ref[5:6, sl]
        x2, y2, z2 = fd_ref[6:7, sl], fd_ref[7:8, sl], fd_ref[8:9, sl]
        e0 = y1 - y2
        e1 = x2 - x1
        e2 = y2 - y0
        e3 = x0 - x2
        denom = e0 * e3 + e1 * (y0 - y2)
        denom_safe = jnp.where(jnp.abs(denom) < 1e-9, 1e-9, denom)
        ic = (lax.broadcasted_iota(jnp.int32, (1, _FC), 1) + c * _FC)
        dok = (jnp.abs(denom) > 1e-9) & (ic < _NF)
        gyrs = [PY - y2 for PY in PYs]
        rows.append((x2, z0, z1, z2, e0, e1, e2, e3,
                     denom_safe, dok, gyrs, ic))

    s_iota = lax.broadcasted_iota(jnp.int32, (_PT, 1), 0)
    for t in range(_ROWS_PER_STEP * _NSUB):
        r = t // _NSUB
        wf = (s_iota + (_PT * (t % _NSUB))).astype(jnp.float32)
        PX = 1.0 - (wf + 0.5) * (2.0 / _IMAGE)
        PXB = jnp.broadcast_to(PX, (_PT, _FC))
        zr = jnp.full((_PT, _FC), jnp.inf, jnp.float32)
        ir = jnp.zeros((_PT, _FC), jnp.int32)
        for c in range(_NCHUNK):
            (x2, z0, z1, z2, e0, e1, e2, e3,
             denom_safe, dok, gyrs, ic) = rows[c]
            gx = PXB - x2
            w0 = (e0 * gx + e1 * gyrs[r]) / denom_safe
            w1 = (e2 * gx + e3 * gyrs[r]) / denom_safe
            w2 = 1.0 - w0 - w1
            zpix = w0 * z0 + w1 * z1 + w2 * z2
            wmin = jnp.minimum(jnp.minimum(w0, w1), w2)
            inside = (wmin >= 0.0) & (zpix > _ZNEAR) & dok
            upd = inside & (zpix < zr)
            zr = jnp.where(upd, zpix, zr)
            ir = jnp.where(upd, jnp.broadcast_to(ic, (_PT, _FC)), ir)
        zmin8 = jnp.min(zr, axis=1, keepdims=True)
        idx8 = jnp.min(jnp.where(zr == zmin8, ir, _FPAD),
                       axis=1, keepdims=True)
        idx_ref[_PT * t:_PT * (t + 1), :] = idx8
        zmin_ref[_PT * t:_PT * (t + 1), :] = zmin8


def _rasterize(vcoord, fidx, interpret=False):
    blk = _ROWS_PER_STEP * _IMAGE
    out_specs = (pl.BlockSpec((blk, 1), lambda g: (g, 0)),
                 pl.BlockSpec((blk, 1), lambda g: (g, 0)),
                 pl.BlockSpec((9, _FPAD), lambda g: (0, 0)))
    return pl.pallas_call(
        _raster_body,
        grid=(_TCSTEPS,),
        in_specs=[pl.BlockSpec((3, _VPAD), lambda g: (0, 0)),
                  pl.BlockSpec((3, _FPAD), lambda g: (0, 0))],
        out_specs=out_specs,
        out_shape=(jax.ShapeDtypeStruct((_NPIX, 1), jnp.int32),
                   jax.ShapeDtypeStruct((_NPIX, 1), jnp.float32),
                   jax.ShapeDtypeStruct((9, _FPAD), jnp.float32)),
        interpret=interpret,
    )(vcoord, fidx)


def _sc_pixel_body(fd_hbm, idx_hbm, zmin_hbm, *refs):
    out_hbm = refs[:6]
    idx_v, z_v, vidx_scr, g_v = refs[6:10]
    outs_v = refs[10:16]
    sem = refs[16]

    wid = lax.axis_index("s") * _NC + lax.axis_index("c")
    base = wid * _PPW
    pltpu.sync_copy(idx_hbm.at[pl.ds(base, _PPW)], idx_v)
    pltpu.sync_copy(zmin_hbm.at[pl.ds(base, _PPW)], z_v)
    for q in range(9):
        for k in range(_PPW // _L):
            iv = idx_v[pl.ds(k * _L, _L)]
            vidx_scr[pl.ds(q * _PPW + k * _L, _L)] = iv + q * _FPAD
    handles = [
        pltpu.async_copy(fd_hbm.at[vidx_scr.at[pl.ds(q * _PPW, _PPW)]],
                         g_v.at[pl.ds(q * _PPW, _PPW)], sem)
        for q in range(9)]
    for h in handles:
        h.wait()

    for k in range(_PPW // _L):
        sl = pl.ds(k * _L, _L)
        x0 = g_v[pl.ds(0 * _PPW + k * _L, _L)]
        y0 = g_v[pl.ds(1 * _PPW + k * _L, _L)]
        z0 = g_v[pl.ds(2 * _PPW + k * _L, _L)]
        x1 = g_v[pl.ds(3 * _PPW + k * _L, _L)]
        y1 = g_v[pl.ds(4 * _PPW + k * _L, _L)]
        z1 = g_v[pl.ds(5 * _PPW + k * _L, _L)]
        x2 = g_v[pl.ds(6 * _PPW + k * _L, _L)]
        y2 = g_v[pl.ds(7 * _PPW + k * _L, _L)]
        iv = idx_v[sl]
        zv = z_v[sl]
        p = base + k * _L + _iota16()
        hh = (p >> 6).astype(jnp.float32)
        ww = (p & 63).astype(jnp.float32)
        PX = 1.0 - (ww + 0.5) * (2.0 / _IMAGE)
        PY = 1.0 - (hh + 0.5) * (2.0 / _IMAGE)

        denom = (y1 - y2) * (x0 - x2) + (x2 - x1) * (y0 - y2)
        denom_safe = jnp.where(jnp.abs(denom) < 1e-9, 1e-9, denom)
        gx = PX - x2
        gy = PY - y2
        w0 = ((y1 - y2) * gx + (x2 - x1) * gy) / denom_safe
        w1 = ((y2 - y0) * gx + (x0 - x2) * gy) / denom_safe
        w2 = 1.0 - w0 - w1
        hit = zv < _BIG

        def seg_d2(ax, ay, bx, by):
            abx, aby = bx - ax, by - ay
            t = jnp.clip(((PX - ax) * abx + (PY - ay) * aby)
                         / (abx * abx + aby * aby + 1e-12), 0.0, 1.0)
            projx, projy = ax + t * abx, ay + t * aby
            rx, ry = PX - projx, PY - projy
            return rx * rx + ry * ry

        d2 = jnp.minimum(jnp.minimum(seg_d2(x0, y0, x1, y1),
                                     seg_d2(x1, y1, x2, y2)),
                         seg_d2(x2, y2, x0, y0))

        outs_v[0][sl] = jnp.where(hit, iv, -1)
        outs_v[1][sl] = jnp.where(hit, zv, -1.0)
        outs_v[2][sl] = jnp.where(hit, w0, -1.0)
        outs_v[3][sl] = jnp.where(hit, w1, -1.0)
        outs_v[4][sl] = jnp.where(hit, w2, -1.0)
        outs_v[5][sl] = jnp.where(hit, -d2, -1.0)

    handles = [
        pltpu.async_copy(outs_v[o], out_hbm[o].at[pl.ds(base, _PPW)], sem)
        for o in range(6)]
    for h in handles:
        h.wait()


def _sc_pixel(fd, idx, zmin):
    mesh = plsc.VectorSubcoreMesh(core_axis_name="c", subcore_axis_name="s")
    otype = [jax.ShapeDtypeStruct((_NPIX,), jnp.int32)]
    otype += [jax.ShapeDtypeStruct((_NPIX,), jnp.float32) for _ in range(5)]
    stypes = [
        pltpu.VMEM((_PPW,), jnp.int32),
        pltpu.VMEM((_PPW,), jnp.float32),
        pltpu.VMEM((9 * _PPW,), jnp.int32),
        pltpu.VMEM((9 * _PPW,), jnp.float32),
        pltpu.VMEM((_PPW,), jnp.int32),
        pltpu.VMEM((_PPW,), jnp.float32),
        pltpu.VMEM((_PPW,), jnp.float32),
        pltpu.VMEM((_PPW,), jnp.float32),
        pltpu.VMEM((_PPW,), jnp.float32),
        pltpu.VMEM((_PPW,), jnp.float32),
        pltpu.SemaphoreType.DMA,
    ]
    f = pl.kernel(
        _sc_pixel_body,
        mesh=mesh,
        out_type=tuple(otype),
        scratch_types=stypes,
    )
    return f(fd, idx, zmin)


def kernel(vertices, faces):
    H = W = _IMAGE
    vcoord = jnp.pad(vertices[0].T, ((0, 0), (0, _VPAD - _NV)))
    fidx = jnp.pad(faces[0].T.astype(jnp.int32), ((0, 0), (0, _FPAD - _NF)),
                   constant_values=_NV)
    idx_r, zmin_r, fd = _rasterize(vcoord, fidx)
    p2f, zbuf, w0m, w1m, w2m, dists = _sc_pixel(
        fd.reshape(-1), idx_r.reshape(_NPIX), zmin_r.reshape(_NPIX))
    shape = (1, H, W)
    p2f = p2f.reshape(shape)
    zbuf = zbuf.reshape(shape)
    bary = jnp.stack([w0m.reshape(shape), w1m.reshape(shape),
                      w2m.reshape(shape)], axis=-1)
    dists = dists.reshape(shape)
    return (p2f[..., None], zbuf[..., None],
            bary[:, :, :, None, :], dists[..., None])


# speedup vs baseline: 1.0802x; 1.0802x over previous
import jax
import jax.numpy as jnp
from jax import lax
from jax.experimental import pallas as pl
from jax.experimental.pallas import tpu as pltpu
from jax.experimental.pallas import tpu_sc as plsc

_IMAGE = 64
_ZNEAR = 0.05
_NF = 2000
_NV = 1200
_FPAD = 2048
_NPIX = _IMAGE * _IMAGE
_NC = 2
_NS = 16
_NW = _NC * _NS
_FPW = _FPAD // _NW
_PPW = _NPIX // _NW
_BIG = 1e38
_L = 16


def _iota16():
    return lax.broadcasted_iota(jnp.int32, (_L,), 0)


def _sc_gather_body(vflat_hbm, fflat_hbm, fd_hbm,
                    fidx_scr, corner_v, vidx_scr, g_v, sem):
    wid = lax.axis_index("s") * _NC + lax.axis_index("c")
    base = wid * _FPW
    lane = _iota16()
    for c3 in range(3):
        for k in range(_FPW // _L):
            fi = 3 * (base + k * _L + lane) + c3
            fidx_scr[pl.ds(c3 * _FPW + k * _L, _L)] = jnp.minimum(
                fi, 3 * _NF - 1)
    handles = [
        pltpu.async_copy(fflat_hbm.at[fidx_scr.at[pl.ds(c3 * _FPW, _FPW)]],
                         corner_v.at[pl.ds(c3 * _FPW, _FPW)], sem)
        for c3 in range(3)]
    for h in handles:
        h.wait()
    for c3 in range(3):
        for k3 in range(3):
            q = c3 * 3 + k3
            for k in range(_FPW // _L):
                iv = corner_v[pl.ds(c3 * _FPW + k * _L, _L)]
                vidx_scr[pl.ds(q * _FPW + k * _L, _L)] = 3 * iv + k3
    handles = [
        pltpu.async_copy(vflat_hbm.at[vidx_scr.at[pl.ds(q * _FPW, _FPW)]],
                         g_v.at[pl.ds(q * _FPW, _FPW)], sem)
        for q in range(9)]
    for h in handles:
        h.wait()
    handles = [
        pltpu.async_copy(g_v.at[pl.ds(q * _FPW, _FPW)],
                         fd_hbm.at[pl.ds(q * _FPAD + base, _FPW)], sem)
        for q in range(9)]
    for h in handles:
        h.wait()


def _sc_gather(vflat, fflat):
    mesh = plsc.VectorSubcoreMesh(core_axis_name="c", subcore_axis_name="s")
    f = pl.kernel(
        _sc_gather_body,
        mesh=mesh,
        out_type=jax.ShapeDtypeStruct((9 * _FPAD,), jnp.float32),
        scratch_types=[
            pltpu.VMEM((3 * _FPW,), jnp.int32),
            pltpu.VMEM((3 * _FPW,), jnp.int32),
            pltpu.VMEM((9 * _FPW,), jnp.int32),
            pltpu.VMEM((9 * _FPW,), jnp.float32),
            pltpu.SemaphoreType.DMA,
        ],
    )
    return f(vflat, fflat)


_FC = 512
_NCHUNK = _FPAD // _FC
_PT = 8
_NSUB = _IMAGE // _PT


_ROWS_PER_STEP = 8
_TCSTEPS = _IMAGE // _ROWS_PER_STEP


def _raster_body(fd_ref, idx_ref, zmin_ref):
    g = pl.program_id(0)
    gf = (_ROWS_PER_STEP * g).astype(jnp.float32)
    PYs = [1.0 - (gf + (r + 0.5)) * (2.0 / _IMAGE)
           for r in range(_ROWS_PER_STEP)]

    rows = []
    for c in range(_NCHUNK):
        sl = pl.ds(c * _FC, _FC)
        x0, y0, z0 = fd_ref[0:1, sl], fd_ref[1:2, sl], fd_ref[2:3, sl]
        x1, y1, z1 = fd_ref[3:4, sl], fd_ref[4:5, sl], fd_ref[5:6, sl]
        x2, y2, z2 = fd_ref[6:7, sl], fd_ref[7:8, sl], fd_ref[8:9, sl]
        e0 = y1 - y2
        e1 = x2 - x1
        e2 = y2 - y0
        e3 = x0 - x2
        denom = e0 * e3 + e1 * (y0 - y2)
        denom_safe = jnp.where(jnp.abs(denom) < 1e-9, 1e-9, denom)
        ic = (lax.broadcasted_iota(jnp.int32, (1, _FC), 1) + c * _FC)
        dok = (jnp.abs(denom) > 1e-9) & (ic < _NF)
        gyrs = [PY - y2 for PY in PYs]
        rows.append((x2, z0, z1, z2, e0, e1, e2, e3,
                     denom_safe, dok, gyrs, ic))

    s_iota = lax.broadcasted_iota(jnp.int32, (_PT, 1), 0)
    for t in range(_ROWS_PER_STEP * _NSUB):
        r = t // _NSUB
        wf = (s_iota + (_PT * (t % _NSUB))).astype(jnp.float32)
        PX = 1.0 - (wf + 0.5) * (2.0 / _IMAGE)
        PXB = jnp.broadcast_to(PX, (_PT, _FC))
        zr = jnp.full((_PT, _FC), jnp.inf, jnp.float32)
        ir = jnp.zeros((_PT, _FC), jnp.int32)
        for c in range(_NCHUNK):
            (x2, z0, z1, z2, e0, e1, e2, e3,
             denom_safe, dok, gyrs, ic) = rows[c]
            gx = PXB - x2
            w0 = (e0 * gx + e1 * gyrs[r]) / denom_safe
            w1 = (e2 * gx + e3 * gyrs[r]) / denom_safe
            w2 = 1.0 - w0 - w1
            zpix = w0 * z0 + w1 * z1 + w2 * z2
            wmin = jnp.minimum(jnp.minimum(w0, w1), w2)
            inside = (wmin >= 0.0) & (zpix > _ZNEAR) & dok
            upd = inside & (zpix < zr)
            zr = jnp.where(upd, zpix, zr)
            ir = jnp.where(upd, jnp.broadcast_to(ic, (_PT, _FC)), ir)
        zmin8 = jnp.min(zr, axis=1, keepdims=True)
        idx8 = jnp.min(jnp.where(zr == zmin8, ir, _FPAD),
                       axis=1, keepdims=True)
        idx_ref[_PT * t:_PT * (t + 1), :] = idx8
        zmin_ref[_PT * t:_PT * (t + 1), :] = zmin8


def _rasterize(fd, interpret=False):
    blk = _ROWS_PER_STEP * _IMAGE
    out_specs = tuple(pl.BlockSpec((blk, 1), lambda g: (g, 0))
                      for _ in range(2))
    return pl.pallas_call(
        _raster_body,
        grid=(_TCSTEPS,),
        in_specs=[pl.BlockSpec((9, _FPAD), lambda g: (0, 0))],
        out_specs=out_specs,
        out_shape=(jax.ShapeDtypeStruct((_NPIX, 1), jnp.int32),
                   jax.ShapeDtypeStruct((_NPIX, 1), jnp.float32)),
        interpret=interpret,
    )(fd)


def _sc_pixel_body(fd_hbm, idx_hbm, zmin_hbm, *refs):
    out_hbm = refs[:6]
    idx_v, z_v, vidx_scr, g_v = refs[6:10]
    outs_v = refs[10:16]
    sem = refs[16]

    wid = lax.axis_index("s") * _NC + lax.axis_index("c")
    base = wid * _PPW
    pltpu.sync_copy(idx_hbm.at[pl.ds(base, _PPW)], idx_v)
    pltpu.sync_copy(zmin_hbm.at[pl.ds(base, _PPW)], z_v)
    for q in range(9):
        for k in range(_PPW // _L):
            iv = idx_v[pl.ds(k * _L, _L)]
            vidx_scr[pl.ds(q * _PPW + k * _L, _L)] = iv + q * _FPAD
    handles = [
        pltpu.async_copy(fd_hbm.at[vidx_scr.at[pl.ds(q * _PPW, _PPW)]],
                         g_v.at[pl.ds(q * _PPW, _PPW)], sem)
        for q in range(9)]
    for h in handles:
        h.wait()

    for k in range(_PPW // _L):
        sl = pl.ds(k * _L, _L)
        x0 = g_v[pl.ds(0 * _PPW + k * _L, _L)]
        y0 = g_v[pl.ds(1 * _PPW + k * _L, _L)]
        z0 = g_v[pl.ds(2 * _PPW + k * _L, _L)]
        x1 = g_v[pl.ds(3 * _PPW + k * _L, _L)]
        y1 = g_v[pl.ds(4 * _PPW + k * _L, _L)]
        z1 = g_v[pl.ds(5 * _PPW + k * _L, _L)]
        x2 = g_v[pl.ds(6 * _PPW + k * _L, _L)]
        y2 = g_v[pl.ds(7 * _PPW + k * _L, _L)]
        iv = idx_v[sl]
        zv = z_v[sl]
        p = base + k * _L + _iota16()
        hh = (p >> 6).astype(jnp.float32)
        ww = (p & 63).astype(jnp.float32)
        PX = 1.0 - (ww + 0.5) * (2.0 / _IMAGE)
        PY = 1.0 - (hh + 0.5) * (2.0 / _IMAGE)

        denom = (y1 - y2) * (x0 - x2) + (x2 - x1) * (y0 - y2)
        denom_safe = jnp.where(jnp.abs(denom) < 1e-9, 1e-9, denom)
        gx = PX - x2
        gy = PY - y2
        w0 = ((y1 - y2) * gx + (x2 - x1) * gy) / denom_safe
        w1 = ((y2 - y0) * gx + (x0 - x2) * gy) / denom_safe
        w2 = 1.0 - w0 - w1
        hit = zv < _BIG

        def seg_d2(ax, ay, bx, by):
            abx, aby = bx - ax, by - ay
            t = jnp.clip(((PX - ax) * abx + (PY - ay) * aby)
                         / (abx * abx + aby * aby + 1e-12), 0.0, 1.0)
            projx, projy = ax + t * abx, ay + t * aby
            rx, ry = PX - projx, PY - projy
            return rx * rx + ry * ry

        d2 = jnp.minimum(jnp.minimum(seg_d2(x0, y0, x1, y1),
                                     seg_d2(x1, y1, x2, y2)),
                         seg_d2(x2, y2, x0, y0))

        outs_v[0][sl] = jnp.where(hit, iv, -1)
        outs_v[1][sl] = jnp.where(hit, zv, -1.0)
        outs_v[2][sl] = jnp.where(hit, w0, -1.0)
        outs_v[3][sl] = jnp.where(hit, w1, -1.0)
        outs_v[4][sl] = jnp.where(hit, w2, -1.0)
        outs_v[5][sl] = jnp.where(hit, -d2, -1.0)

    handles = [
        pltpu.async_copy(outs_v[o], out_hbm[o].at[pl.ds(base, _PPW)], sem)
        for o in range(6)]
    for h in handles:
        h.wait()


def _sc_pixel(fd, idx, zmin):
    mesh = plsc.VectorSubcoreMesh(core_axis_name="c", subcore_axis_name="s")
    otype = [jax.ShapeDtypeStruct((_NPIX,), jnp.int32)]
    otype += [jax.ShapeDtypeStruct((_NPIX,), jnp.float32) for _ in range(5)]
    stypes = [
        pltpu.VMEM((_PPW,), jnp.int32),
        pltpu.VMEM((_PPW,), jnp.float32),
        pltpu.VMEM((9 * _PPW,), jnp.int32),
        pltpu.VMEM((9 * _PPW,), jnp.float32),
        pltpu.VMEM((_PPW,), jnp.int32),
        pltpu.VMEM((_PPW,), jnp.float32),
        pltpu.VMEM((_PPW,), jnp.float32),
        pltpu.VMEM((_PPW,), jnp.float32),
        pltpu.VMEM((_PPW,), jnp.float32),
        pltpu.VMEM((_PPW,), jnp.float32),
        pltpu.SemaphoreType.DMA,
    ]
    f = pl.kernel(
        _sc_pixel_body,
        mesh=mesh,
        out_type=tuple(otype),
        scratch_types=stypes,
    )
    return f(fd, idx, zmin)


def kernel(vertices, faces):
    H = W = _IMAGE
    vflat = vertices.reshape(-1)
    fflat = faces.reshape(-1).astype(jnp.int32)
    fd = _sc_gather(vflat, fflat)
    idx_r, zmin_r = _rasterize(fd.reshape(9, _FPAD))
    p2f, zbuf, w0m, w1m, w2m, dists = _sc_pixel(
        fd, idx_r.reshape(_NPIX), zmin_r.reshape(_NPIX))
    shape = (1, H, W)
    p2f = p2f.reshape(shape)
    zbuf = zbuf.reshape(shape)
    bary = jnp.stack([w0m.reshape(shape), w1m.reshape(shape),
                      w2m.reshape(shape)], axis=-1)
    dists = dists.reshape(shape)
    return (p2f[..., None], zbuf[..., None],
            bary[:, :, :, None, :], dists[..., None])


# speedup vs baseline: 1.0920x; 1.0109x over previous
import jax
import jax.numpy as jnp
from jax import lax
from jax.experimental import pallas as pl
from jax.experimental.pallas import tpu as pltpu
from jax.experimental.pallas import tpu_sc as plsc

_IMAGE = 64
_ZNEAR = 0.05
_NF = 2000
_NV = 1200
_FPAD = 2048
_NPIX = _IMAGE * _IMAGE
_NC = 2
_NS = 16
_NW = _NC * _NS
_FPW = _FPAD // _NW
_PPW = _NPIX // _NW
_BIG = 1e38
_L = 16


def _iota16():
    return lax.broadcasted_iota(jnp.int32, (_L,), 0)


def _sc_gather_body(vflat_hbm, fflat_hbm, fd_hbm,
                    fidx_scr, corner_v, vidx_scr, g_v, sem):
    wid = lax.axis_index("s") * _NC + lax.axis_index("c")
    base = wid * _FPW
    lane = _iota16()
    for c3 in range(3):
        for k in range(_FPW // _L):
            fi = 3 * (base + k * _L + lane) + c3
            fidx_scr[pl.ds(c3 * _FPW + k * _L, _L)] = jnp.minimum(
                fi, 3 * _NF - 1)
    handles = [
        pltpu.async_copy(fflat_hbm.at[fidx_scr.at[pl.ds(c3 * _FPW, _FPW)]],
                         corner_v.at[pl.ds(c3 * _FPW, _FPW)], sem)
        for c3 in range(3)]
    for h in handles:
        h.wait()
    for c3 in range(3):
        for k3 in range(3):
            q = c3 * 3 + k3
            for k in range(_FPW // _L):
                iv = corner_v[pl.ds(c3 * _FPW + k * _L, _L)]
                vidx_scr[pl.ds(q * _FPW + k * _L, _L)] = 3 * iv + k3
    handles = [
        pltpu.async_copy(vflat_hbm.at[vidx_scr.at[pl.ds(q * _FPW, _FPW)]],
                         g_v.at[pl.ds(q * _FPW, _FPW)], sem)
        for q in range(9)]
    for h in handles:
        h.wait()
    handles = [
        pltpu.async_copy(g_v.at[pl.ds(q * _FPW, _FPW)],
                         fd_hbm.at[pl.ds(q * _FPAD + base, _FPW)], sem)
        for q in range(9)]
    for h in handles:
        h.wait()


def _sc_gather(vflat, fflat):
    mesh = plsc.VectorSubcoreMesh(core_axis_name="c", subcore_axis_name="s")
    f = pl.kernel(
        _sc_gather_body,
        mesh=mesh,
        out_type=jax.ShapeDtypeStruct((9 * _FPAD,), jnp.float32),
        scratch_types=[
            pltpu.VMEM((3 * _FPW,), jnp.int32),
            pltpu.VMEM((3 * _FPW,), jnp.int32),
            pltpu.VMEM((9 * _FPW,), jnp.int32),
            pltpu.VMEM((9 * _FPW,), jnp.float32),
            pltpu.SemaphoreType.DMA,
        ],
    )
    return f(vflat, fflat)


_FC = 256
_NCHUNK = _FPAD // _FC
_PT = 8
_NSUB = _IMAGE // _PT


_ROWS_PER_STEP = 8
_TCSTEPS = _IMAGE // _ROWS_PER_STEP


def _raster_body(fd_ref, idx_ref, zmin_ref):
    g = pl.program_id(0)
    gf = (_ROWS_PER_STEP * g).astype(jnp.float32)
    PYs = [1.0 - (gf + (r + 0.5)) * (2.0 / _IMAGE)
           for r in range(_ROWS_PER_STEP)]

    rows = []
    for c in range(_NCHUNK):
        sl = pl.ds(c * _FC, _FC)
        x0, y0, z0 = fd_ref[0:1, sl], fd_ref[1:2, sl], fd_ref[2:3, sl]
        x1, y1, z1 = fd_ref[3:4, sl], fd_ref[4:5, sl], fd_ref[5:6, sl]
        x2, y2, z2 = fd_ref[6:7, sl], fd_ref[7:8, sl], fd_ref[8:9, sl]
        e0 = y1 - y2
        e1 = x2 - x1
        e2 = y2 - y0
        e3 = x0 - x2
        denom = e0 * e3 + e1 * (y0 - y2)
        denom_safe = jnp.where(jnp.abs(denom) < 1e-9, 1e-9, denom)
        ic = (lax.broadcasted_iota(jnp.int32, (1, _FC), 1) + c * _FC)
        dok = (jnp.abs(denom) > 1e-9) & (ic < _NF)
        gyrs = [PY - y2 for PY in PYs]
        rows.append((x2, z0, z1, z2, e0, e1, e2, e3,
                     denom_safe, dok, gyrs, ic))

    s_iota = lax.broadcasted_iota(jnp.int32, (_PT, 1), 0)
    for t in range(_ROWS_PER_STEP * _NSUB):
        r = t // _NSUB
        wf = (s_iota + (_PT * (t % _NSUB))).astype(jnp.float32)
        PX = 1.0 - (wf + 0.5) * (2.0 / _IMAGE)
        PXB = jnp.broadcast_to(PX, (_PT, _FC))
        zr = jnp.full((_PT, _FC), jnp.inf, jnp.float32)
        ir = jnp.zeros((_PT, _FC), jnp.int32)
        for c in range(_NCHUNK):
            (x2, z0, z1, z2, e0, e1, e2, e3,
             denom_safe, dok, gyrs, ic) = rows[c]
            gx = PXB - x2
            w0 = (e0 * gx + e1 * gyrs[r]) / denom_safe
            w1 = (e2 * gx + e3 * gyrs[r]) / denom_safe
            w2 = 1.0 - w0 - w1
            zpix = w0 * z0 + w1 * z1 + w2 * z2
            wmin = jnp.minimum(jnp.minimum(w0, w1), w2)
            inside = (wmin >= 0.0) & (zpix > _ZNEAR) & dok
            upd = inside & (zpix < zr)
            zr = jnp.where(upd, zpix, zr)
            ir = jnp.where(upd, jnp.broadcast_to(ic, (_PT, _FC)), ir)
        zmin8 = jnp.min(zr, axis=1, keepdims=True)
        idx8 = jnp.min(jnp.where(zr == zmin8, ir, _FPAD),
                       axis=1, keepdims=True)
        idx_ref[_PT * t:_PT * (t + 1), :] = idx8
        zmin_ref[_PT * t:_PT * (t + 1), :] = zmin8


def _rasterize(fd):
    blk = _ROWS_PER_STEP * _IMAGE
    out_specs = tuple(pl.BlockSpec((blk, 1), lambda g: (g, 0))
                      for _ in range(2))
    return pl.pallas_call(
        _raster_body,
        grid=(_TCSTEPS,),
        in_specs=[pl.BlockSpec((9, _FPAD), lambda g: (0, 0))],
        out_specs=out_specs,
        out_shape=(jax.ShapeDtypeStruct((_NPIX, 1), jnp.int32),
                   jax.ShapeDtypeStruct((_NPIX, 1), jnp.float32)),
    )(fd)


def _sc_pixel_body(fd_hbm, idx_hbm, zmin_hbm, *refs):
    out_hbm = refs[:6]
    idx_v, z_v, vidx_scr, g_v = refs[6:10]
    outs_v = refs[10:16]
    sem = refs[16]

    wid = lax.axis_index("s") * _NC + lax.axis_index("c")
    base = wid * _PPW
    pltpu.sync_copy(idx_hbm.at[pl.ds(base, _PPW)], idx_v)
    pltpu.sync_copy(zmin_hbm.at[pl.ds(base, _PPW)], z_v)
    for q in range(9):
        for k in range(_PPW // _L):
            iv = idx_v[pl.ds(k * _L, _L)]
            vidx_scr[pl.ds(q * _PPW + k * _L, _L)] = iv + q * _FPAD
    handles = [
        pltpu.async_copy(fd_hbm.at[vidx_scr.at[pl.ds(q * _PPW, _PPW)]],
                         g_v.at[pl.ds(q * _PPW, _PPW)], sem)
        for q in range(9)]
    for h in handles:
        h.wait()

    for k in range(_PPW // _L):
        sl = pl.ds(k * _L, _L)
        x0 = g_v[pl.ds(0 * _PPW + k * _L, _L)]
        y0 = g_v[pl.ds(1 * _PPW + k * _L, _L)]
        z0 = g_v[pl.ds(2 * _PPW + k * _L, _L)]
        x1 = g_v[pl.ds(3 * _PPW + k * _L, _L)]
        y1 = g_v[pl.ds(4 * _PPW + k * _L, _L)]
        z1 = g_v[pl.ds(5 * _PPW + k * _L, _L)]
        x2 = g_v[pl.ds(6 * _PPW + k * _L, _L)]
        y2 = g_v[pl.ds(7 * _PPW + k * _L, _L)]
        iv = idx_v[sl]
        zv = z_v[sl]
        p = base + k * _L + _iota16()
        hh = (p >> 6).astype(jnp.float32)
        ww = (p & 63).astype(jnp.float32)
        PX = 1.0 - (ww + 0.5) * (2.0 / _IMAGE)
        PY = 1.0 - (hh + 0.5) * (2.0 / _IMAGE)

        denom = (y1 - y2) * (x0 - x2) + (x2 - x1) * (y0 - y2)
        denom_safe = jnp.where(jnp.abs(denom) < 1e-9, 1e-9, denom)
        gx = PX - x2
        gy = PY - y2
        w0 = ((y1 - y2) * gx + (x2 - x1) * gy) / denom_safe
        w1 = ((y2 - y0) * gx + (x0 - x2) * gy) / denom_safe
        w2 = 1.0 - w0 - w1
        hit = zv < _BIG

        def seg_d2(ax, ay, bx, by):
            abx, aby = bx - ax, by - ay
            t = jnp.clip(((PX - ax) * abx + (PY - ay) * aby)
                         / (abx * abx + aby * aby + 1e-12), 0.0, 1.0)
            projx, projy = ax + t * abx, ay + t * aby
            rx, ry = PX - projx, PY - projy
            return rx * rx + ry * ry

        d2 = jnp.minimum(jnp.minimum(seg_d2(x0, y0, x1, y1),
                                     seg_d2(x1, y1, x2, y2)),
                         seg_d2(x2, y2, x0, y0))

        outs_v[0][sl] = jnp.where(hit, iv, -1)
        outs_v[1][sl] = jnp.where(hit, zv, -1.0)
        outs_v[2][sl] = jnp.where(hit, w0, -1.0)
        outs_v[3][sl] = jnp.where(hit, w1, -1.0)
        outs_v[4][sl] = jnp.where(hit, w2, -1.0)
        outs_v[5][sl] = jnp.where(hit, -d2, -1.0)

    handles = [
        pltpu.async_copy(outs_v[o], out_hbm[o].at[pl.ds(base, _PPW)], sem)
        for o in range(6)]
    for h in handles:
        h.wait()


def _sc_pixel(fd, idx, zmin):
    mesh = plsc.VectorSubcoreMesh(core_axis_name="c", subcore_axis_name="s")
    otype = [jax.ShapeDtypeStruct((_NPIX,), jnp.int32)]
    otype += [jax.ShapeDtypeStruct((_NPIX,), jnp.float32) for _ in range(5)]
    stypes = [
        pltpu.VMEM((_PPW,), jnp.int32),
        pltpu.VMEM((_PPW,), jnp.float32),
        pltpu.VMEM((9 * _PPW,), jnp.int32),
        pltpu.VMEM((9 * _PPW,), jnp.float32),
        pltpu.VMEM((_PPW,), jnp.int32),
        pltpu.VMEM((_PPW,), jnp.float32),
        pltpu.VMEM((_PPW,), jnp.float32),
        pltpu.VMEM((_PPW,), jnp.float32),
        pltpu.VMEM((_PPW,), jnp.float32),
        pltpu.VMEM((_PPW,), jnp.float32),
        pltpu.SemaphoreType.DMA,
    ]
    f = pl.kernel(
        _sc_pixel_body,
        mesh=mesh,
        out_type=tuple(otype),
        scratch_types=stypes,
    )
    return f(fd, idx, zmin)


def kernel(vertices, faces):
    H = W = _IMAGE
    vflat = vertices.reshape(-1)
    fflat = faces.reshape(-1).astype(jnp.int32)
    fd = _sc_gather(vflat, fflat)
    idx_r, zmin_r = _rasterize(fd.reshape(9, _FPAD))
    p2f, zbuf, w0m, w1m, w2m, dists = _sc_pixel(
        fd, idx_r.reshape(_NPIX), zmin_r.reshape(_NPIX))
    shape = (1, H, W)
    p2f = p2f.reshape(shape)
    zbuf = zbuf.reshape(shape)
    bary = jnp.stack([w0m.reshape(shape), w1m.reshape(shape),
                      w2m.reshape(shape)], axis=-1)
    dists = dists.reshape(shape)
    return (p2f[..., None], zbuf[..., None],
            bary[:, :, :, None, :], dists[..., None])
